# Initial kernel scaffold; baseline (speedup 1.0000x reference)
import jax
import jax.numpy as jnp
from jax.experimental import pallas as pl


def kernel(x, edge_index, edge_attr, batch, emb_tables, W1, b1, W2, b2, W3, b3, bn_gamma, bn_beta, Wih, Whh, bih, bhh, lin1_W, lin1_b, lin2_W, lin2_b):
    raise NotImplementedError("write your pallas kernel here")


# speedup vs baseline: 2.0106x; 2.0106x over previous
import functools

import jax
import jax.numpy as jnp
from jax.experimental import pallas as pl
from jax.experimental.pallas import tpu as pltpu


def _mm_kernel(a_ref, b_ref, o_ref):
    o_ref[...] = jnp.dot(a_ref[...], b_ref[...],
                         preferred_element_type=jnp.float32)


def _mm(a, b):
    m, k = a.shape
    k2, n = b.shape
    return pl.pallas_call(
        _mm_kernel,
        out_shape=jax.ShapeDtypeStruct((m, n), jnp.float32),
    )(a, b)


def _bn_relu_kernel(h_ref, g_ref, b_ref, o_ref, *, relu_first):
    h = h_ref[...]
    if relu_first:
        h = jnp.maximum(h, 0.0)
    mu = jnp.mean(h, axis=0, keepdims=True)
    var = jnp.mean((h - mu) ** 2, axis=0, keepdims=True)
    o_ref[...] = (h - mu) / jnp.sqrt(var + 1e-5) * g_ref[...] + b_ref[...]


def _bn_relu(h, gamma, beta):
    return pl.pallas_call(
        functools.partial(_bn_relu_kernel, relu_first=True),
        out_shape=jax.ShapeDtypeStruct(h.shape, jnp.float32),
    )(h, gamma[None, :], beta[None, :])


def kernel(x, edge_index, edge_attr, batch, emb_tables, W1, b1, W2, b2, W3, b3,
           bn_gamma, bn_beta, Wih, Whh, bih, bhh, lin1_W, lin1_b, lin2_W, lin2_b):
    del edge_attr
    n = x.shape[0]
    n_graphs = 256
    emb = emb_tables[0][x[:, 0]]
    for i in range(1, len(emb_tables)):
        emb = emb + emb_tables[i][x[:, i]]
    src = edge_index[0]
    dst = edge_index[1]
    deg = jnp.ones((n,), jnp.float32).at[dst].add(1.0)
    dinv = 1.0 / jnp.sqrt(deg)

    def conv2(h, W, b):
        m = _mm(h, W.T)
        ms = m * dinv[:, None]
        s = jnp.zeros_like(ms).at[dst].add(ms[src])
        return dinv[:, None] * (s + ms) + b

    out = _bn_relu(conv2(emb, W1, b1), bn_gamma, bn_beta)
    out = _bn_relu(conv2(out, W2, b2), bn_gamma, bn_beta)
    out = conv2(out, W3, b3)

    d = out.shape[1]
    h = jnp.zeros((n_graphs, d), jnp.float32)
    c = jnp.zeros((n_graphs, d), jnp.float32)
    q_star = jnp.zeros((n_graphs, 2 * d), jnp.float32)
    for _ in range(4):
        g = _mm(q_star, Wih.T) + bih + _mm(h, Whh.T) + bhh
        i_g, f_g, g_g, o_g = jnp.split(g, 4, axis=-1)
        c = jax.nn.sigmoid(f_g) * c + jax.nn.sigmoid(i_g) * jnp.tanh(g_g)
        h = jax.nn.sigmoid(o_g) * jnp.tanh(c)
        q = h
        e = jnp.sum(out * q[batch], axis=-1)
        emax = jax.ops.segment_max(e, batch, num_segments=n_graphs)
        emax = jnp.where(jnp.isfinite(emax), emax, 0.0)
        ee = jnp.exp(e - emax[batch])
        den = jax.ops.segment_sum(ee, batch, num_segments=n_graphs)
        a = ee / (den[batch] + 1e-16)
        r = jax.ops.segment_sum(a[:, None] * out, batch, num_segments=n_graphs)
        q_star = jnp.concatenate([q, r], axis=-1)
    z = _mm(q_star, lin1_W.T) + lin1_b
    z = _mm(z, lin2_W.T) + lin2_b
    return jax.nn.sigmoid(z)


# speedup vs baseline: 6.2148x; 3.0910x over previous
import functools

import jax
import jax.numpy as jnp
from jax import lax
from jax.experimental import pallas as pl
from jax.experimental.pallas import tpu as pltpu
from jax.experimental.pallas import tpu_sc as plsc

_N = 10000
_NP = 10240
_E = 320000
_H = 128
_G = 256
_W = 128
_NWIN = 79
_EPW = _W * _NWIN
_NWORK = 32
_EP = _EPW * _NWORK
_RPS = _NP // 16

_mesh = plsc.VectorSubcoreMesh(core_axis_name="core", subcore_axis_name="subcore")


def _sc_conv_kernel(ms_hbm, src_hbm, dst_hbm, out_hbm,
                    src_v, dst_v, rows_v, zer_v, acc_sh):
    cid = lax.axis_index("core")
    sid = lax.axis_index("subcore")
    wid = cid * 16 + sid

    @pl.loop(0, 64)
    def _(r):
        @pl.loop(0, _H, step=16)
        def _(c):
            zer_v.at[pl.ds(r, 1), pl.ds(c, 16)][...] = jnp.zeros(
                (1, 16), jnp.float32)

    @pl.loop(0, _RPS // 64)
    def _(i):
        pltpu.sync_copy(zer_v, acc_sh.at[pl.ds(sid * _RPS + i * 64, 64)])

    plsc.subcore_barrier()

    pltpu.sync_copy(src_hbm.at[wid], src_v)
    pltpu.sync_copy(dst_hbm.at[wid], dst_v)

    @pl.loop(0, _NWIN)
    def _(w):
        pltpu.sync_copy(ms_hbm.at[src_v.at[w]], rows_v)
        pltpu.sync_copy(rows_v, acc_sh.at[dst_v.at[w]], add=True)

    plsc.subcore_barrier()
    pltpu.sync_copy(acc_sh.at[pl.ds(sid * _RPS, _RPS)],
                    out_hbm.at[cid, pl.ds(sid * _RPS, _RPS)])


def _sc_conv(ms_pad, src_w, dst_w):
    return pl.kernel(
        _sc_conv_kernel,
        out_type=jax.ShapeDtypeStruct((2, _NP, _H), jnp.float32),
        mesh=_mesh,
        scratch_types=[
            pltpu.VMEM((_NWIN, _W), jnp.int32),
            pltpu.VMEM((_NWIN, _W), jnp.int32),
            pltpu.VMEM((_W, _H), jnp.float32),
            pltpu.VMEM((64, _H), jnp.float32),
            pltpu.VMEM_SHARED((_NP, _H), jnp.float32),
        ],
    )(ms_pad, src_w, dst_w)


def _sc_deg_kernel(dst_hbm, out_hbm, dst_v, ones_v, acc_sh):
    cid = lax.axis_index("core")
    sid = lax.axis_index("subcore")
    wid = cid * 16 + sid

    @pl.loop(0, _W)
    def _(r):
        ones_v.at[pl.ds(r, 1), pl.ds(0, 16)][...] = jnp.ones((1, 16),
                                                             jnp.float32)

    @pl.loop(0, _RPS // _W)
    def _(i):
        pltpu.sync_copy(ones_v, acc_sh.at[pl.ds(sid * _RPS + i * _W, _W)])

    plsc.subcore_barrier()

    pltpu.sync_copy(dst_hbm.at[wid], dst_v)

    @pl.loop(0, _NWIN)
    def _(w):
        pltpu.sync_copy(ones_v, acc_sh.at[dst_v.at[w]], add=True)

    plsc.subcore_barrier()
    pltpu.sync_copy(acc_sh.at[pl.ds(sid * _RPS, _RPS)],
                    out_hbm.at[cid, pl.ds(sid * _RPS, _RPS)])


def _sc_deg(dst_w):
    return pl.kernel(
        _sc_deg_kernel,
        out_type=jax.ShapeDtypeStruct((2, _NP, 16), jnp.float32),
        mesh=_mesh,
        scratch_types=[
            pltpu.VMEM((_NWIN, _W), jnp.int32),
            pltpu.VMEM((_W, 16), jnp.float32),
            pltpu.VMEM_SHARED((_NP, 16), jnp.float32),
        ],
    )(dst_w)


def _mm_kernel(a_ref, b_ref, o_ref):
    o_ref[...] = jnp.dot(a_ref[...], b_ref[...],
                         preferred_element_type=jnp.float32)


def _mm(a, b):
    m, _ = a.shape
    _, n = b.shape
    return pl.pallas_call(
        _mm_kernel,
        out_shape=jax.ShapeDtypeStruct((m, n), jnp.float32),
    )(a, b)


def _bn_relu_kernel(h_ref, g_ref, b_ref, o_ref, *, relu_first):
    h = h_ref[...]
    if relu_first:
        h = jnp.maximum(h, 0.0)
    mu = jnp.mean(h, axis=0, keepdims=True)
    var = jnp.mean((h - mu) ** 2, axis=0, keepdims=True)
    o_ref[...] = (h - mu) / jnp.sqrt(var + 1e-5) * g_ref[...] + b_ref[...]


def _bn_relu(h, gamma, beta):
    return pl.pallas_call(
        functools.partial(_bn_relu_kernel, relu_first=True),
        out_shape=jax.ShapeDtypeStruct(h.shape, jnp.float32),
    )(h, gamma[None, :], beta[None, :])


def kernel(x, edge_index, edge_attr, batch, emb_tables, W1, b1, W2, b2, W3, b3,
           bn_gamma, bn_beta, Wih, Whh, bih, bhh, lin1_W, lin1_b, lin2_W, lin2_b):
    del edge_attr
    emb = emb_tables[0][x[:, 0]]
    for i in range(1, len(emb_tables)):
        emb = emb + emb_tables[i][x[:, i]]

    src = edge_index[0].astype(jnp.int32)
    dst = edge_index[1].astype(jnp.int32)
    pad = _N + (jnp.arange(_EP - _E, dtype=jnp.int32) % (_NP - _N))
    src_w = jnp.concatenate([src, pad]).reshape(_NWORK, _NWIN, _W)
    dst_w = jnp.concatenate([dst, pad]).reshape(_NWORK, _NWIN, _W)

    deg_parts = _sc_deg(dst_w)
    deg = deg_parts[0, :_N, 0] + deg_parts[1, :_N, 0] - 1.0
    dinv = lax.rsqrt(deg)

    def conv(h, W, b):
        ms = _mm(h, W.T) * dinv[:, None]
        ms_pad = jnp.pad(ms, ((0, _NP - _N), (0, 0)))
        parts = _sc_conv(ms_pad, src_w, dst_w)
        s = parts[0, :_N] + parts[1, :_N]
        return dinv[:, None] * (s + ms) + b

    out = _bn_relu(conv(emb, W1, b1), bn_gamma, bn_beta)
    out = _bn_relu(conv(out, W2, b2), bn_gamma, bn_beta)
    out = conv(out, W3, b3)

    d = out.shape[1]
    h = jnp.zeros((_G, d), jnp.float32)
    c = jnp.zeros((_G, d), jnp.float32)
    q_star = jnp.zeros((_G, 2 * d), jnp.float32)
    for _ in range(4):
        g = _mm(q_star, Wih.T) + bih + _mm(h, Whh.T) + bhh
        i_g, f_g, g_g, o_g = jnp.split(g, 4, axis=-1)
        c = jax.nn.sigmoid(f_g) * c + jax.nn.sigmoid(i_g) * jnp.tanh(g_g)
        h = jax.nn.sigmoid(o_g) * jnp.tanh(c)
        q = h
        e = jnp.sum(out * q[batch], axis=-1)
        emax = jax.ops.segment_max(e, batch, num_segments=_G)
        emax = jnp.where(jnp.isfinite(emax), emax, 0.0)
        ee = jnp.exp(e - emax[batch])
        den = jax.ops.segment_sum(ee, batch, num_segments=_G)
        a = ee / (den[batch] + 1e-16)
        r = jax.ops.segment_sum(a[:, None] * out, batch, num_segments=_G)
        q_star = jnp.concatenate([q, r], axis=-1)
    z = _mm(q_star, lin1_W.T) + lin1_b
    z = _mm(z, lin2_W.T) + lin2_b
    return jax.nn.sigmoid(z)


# speedup vs baseline: 17.7651x; 2.8585x over previous
import functools

import jax
import jax.numpy as jnp
from jax import lax
from jax.experimental import pallas as pl
from jax.experimental.pallas import tpu as pltpu
from jax.experimental.pallas import tpu_sc as plsc

_N = 10000
_NP = 10240
_E = 320000
_H = 128
_G = 256
_W = 128
_NWIN = 79
_NWORK = 32
_EP = _W * _NWIN * _NWORK
_RPS = _NP // 16
_FDIMS = (119, 4, 12, 12, 10, 6, 6, 2, 2)
_FTOT = sum(_FDIMS)

@functools.cache
def _mesh():
    return plsc.VectorSubcoreMesh(core_axis_name="core",
                                  subcore_axis_name="subcore")


def _sc_conv_kernel(ms_hbm, src_hbm, dst_hbm, out_hbm,
                    src_v, dst_v, rows_v, zer_v, acc_sh):
    cid = lax.axis_index("core")
    sid = lax.axis_index("subcore")
    wid = cid * 16 + sid

    @pl.loop(0, 64)
    def _(r):
        @pl.loop(0, _H, step=16)
        def _(c):
            zer_v.at[pl.ds(r, 1), pl.ds(c, 16)][...] = jnp.zeros(
                (1, 16), jnp.float32)

    @pl.loop(0, _RPS // 64)
    def _(i):
        pltpu.sync_copy(zer_v, acc_sh.at[pl.ds(sid * _RPS + i * 64, 64)])

    plsc.subcore_barrier()

    pltpu.sync_copy(src_hbm.at[wid], src_v)
    pltpu.sync_copy(dst_hbm.at[wid], dst_v)

    @pl.loop(0, _NWIN)
    def _(w):
        pltpu.sync_copy(ms_hbm.at[src_v.at[w]], rows_v)
        pltpu.sync_copy(rows_v, acc_sh.at[dst_v.at[w]], add=True)

    plsc.subcore_barrier()
    pltpu.sync_copy(acc_sh.at[pl.ds(sid * _RPS, _RPS)],
                    out_hbm.at[cid, pl.ds(sid * _RPS, _RPS)])


def _sc_conv(ms_pad, src_w, dst_w):
    return pl.kernel(
        _sc_conv_kernel,
        out_type=jax.ShapeDtypeStruct((2, _NP, _H), jnp.float32),
        mesh=_mesh(),
        scratch_types=[
            pltpu.VMEM((_NWIN, _W), jnp.int32),
            pltpu.VMEM((_NWIN, _W), jnp.int32),
            pltpu.VMEM((_W, _H), jnp.float32),
            pltpu.VMEM((64, _H), jnp.float32),
            pltpu.VMEM_SHARED((_NP, _H), jnp.float32),
        ],
    )(ms_pad, src_w, dst_w)


def _sc_deg_kernel(dst_hbm, out_hbm, dst_v, ones_v, acc_sh):
    cid = lax.axis_index("core")
    sid = lax.axis_index("subcore")
    wid = cid * 16 + sid

    @pl.loop(0, _W)
    def _(r):
        ones_v.at[pl.ds(r, 1), pl.ds(0, 16)][...] = jnp.ones((1, 16),
                                                             jnp.float32)

    @pl.loop(0, _RPS // _W)
    def _(i):
        pltpu.sync_copy(ones_v, acc_sh.at[pl.ds(sid * _RPS + i * _W, _W)])

    plsc.subcore_barrier()

    pltpu.sync_copy(dst_hbm.at[wid], dst_v)

    @pl.loop(0, _NWIN)
    def _(w):
        pltpu.sync_copy(ones_v, acc_sh.at[dst_v.at[w]], add=True)

    plsc.subcore_barrier()
    pltpu.sync_copy(acc_sh.at[pl.ds(sid * _RPS, _RPS)],
                    out_hbm.at[cid, pl.ds(sid * _RPS, _RPS)])


def _sc_deg(dst_w):
    return pl.kernel(
        _sc_deg_kernel,
        out_type=jax.ShapeDtypeStruct((2, _NP, 16), jnp.float32),
        mesh=_mesh(),
        scratch_types=[
            pltpu.VMEM((_NWIN, _W), jnp.int32),
            pltpu.VMEM((_W, 16), jnp.float32),
            pltpu.VMEM_SHARED((_NP, 16), jnp.float32),
        ],
    )(dst_w)


def _emb_kernel(x_ref, t_ref, o_ref):
    cols = lax.broadcasted_iota(jnp.int32, (_N, _FTOT), 1)
    mh = jnp.zeros((_N, _FTOT), jnp.float32)
    off = 0
    for f, d in enumerate(_FDIMS):
        mh = mh + (cols == x_ref[:, f:f + 1] + off).astype(jnp.float32)
        off += d
    o_ref[...] = jnp.dot(mh, t_ref[...], preferred_element_type=jnp.float32)


def _emb(x, tables_cat):
    return pl.pallas_call(
        _emb_kernel,
        out_shape=jax.ShapeDtypeStruct((_N, _H), jnp.float32),
    )(x, tables_cat)


def _pre_kernel(emb_ref, degp_ref, w_ref, o_ref, dinv_ref):
    deg = degp_ref[0, :_N, 0:1] + degp_ref[1, :_N, 0:1] - 1.0
    dinv = lax.rsqrt(deg)
    dinv_ref[...] = dinv
    ms = jnp.dot(emb_ref[...], w_ref[...],
                 preferred_element_type=jnp.float32) * dinv
    o_ref[:_N, :] = ms
    o_ref[_N:, :] = jnp.zeros((_NP - _N, _H), jnp.float32)


def _pre(emb, deg_parts, W1):
    return pl.pallas_call(
        _pre_kernel,
        out_shape=(jax.ShapeDtypeStruct((_NP, _H), jnp.float32),
                   jax.ShapeDtypeStruct((_N, 1), jnp.float32)),
    )(emb, deg_parts, W1.T)


def _post_kernel(p_ref, ms_ref, dinv_ref, b_ref, g_ref, bt_ref, w_ref, o_ref):
    dinv = dinv_ref[...]
    s = p_ref[0, :_N, :] + p_ref[1, :_N, :] + ms_ref[:_N, :]
    h = dinv * s + b_ref[...]
    h = jnp.maximum(h, 0.0)
    mu = jnp.mean(h, axis=0, keepdims=True)
    var = jnp.mean((h - mu) ** 2, axis=0, keepdims=True)
    h = (h - mu) / jnp.sqrt(var + 1e-5) * g_ref[...] + bt_ref[...]
    ms = jnp.dot(h, w_ref[...], preferred_element_type=jnp.float32) * dinv
    o_ref[:_N, :] = ms
    o_ref[_N:, :] = jnp.zeros((_NP - _N, _H), jnp.float32)


def _post(parts, ms_pad, dinv, b, gamma, beta, W_next):
    return pl.pallas_call(
        _post_kernel,
        out_shape=jax.ShapeDtypeStruct((_NP, _H), jnp.float32),
    )(parts, ms_pad, dinv, b[None, :], gamma[None, :], beta[None, :], W_next.T)


def _out3_kernel(p_ref, ms_ref, dinv_ref, b_ref, o_ref):
    s = p_ref[0, :_N, :] + p_ref[1, :_N, :] + ms_ref[:_N, :]
    o_ref[...] = dinv_ref[...] * s + b_ref[...]


def _out3(parts, ms_pad, dinv, b):
    return pl.pallas_call(
        _out3_kernel,
        out_shape=jax.ShapeDtypeStruct((_N, _H), jnp.float32),
    )(parts, ms_pad, dinv, b[None, :])


def _onehot_kernel(batch_ref, o_ref):
    o_ref[...] = (batch_ref[...] == lax.broadcasted_iota(
        jnp.int32, (_N, _G), 1)).astype(jnp.float32)


def _onehot(batch):
    return pl.pallas_call(
        _onehot_kernel,
        out_shape=jax.ShapeDtypeStruct((_N, _G), jnp.float32),
    )(batch[:, None].astype(jnp.int32))


def _s2s_step_kernel(oh_ref, out_ref, h_ref, c_ref, qs_ref,
                     wih_ref, whh_ref, bih_ref, bhh_ref,
                     h_o, c_o, qs_o):
    onehot = oh_ref[...]
    out = out_ref[...]
    g = (jnp.dot(qs_ref[...], wih_ref[...], preferred_element_type=jnp.float32)
         + bih_ref[...]
         + jnp.dot(h_ref[...], whh_ref[...], preferred_element_type=jnp.float32)
         + bhh_ref[...])
    i_g, f_g, g_g, o_g = jnp.split(g, 4, axis=-1)
    c = jax.nn.sigmoid(f_g) * c_ref[...] + jax.nn.sigmoid(i_g) * jnp.tanh(g_g)
    h = jax.nn.sigmoid(o_g) * jnp.tanh(c)
    qb = jnp.dot(onehot, h, preferred_element_type=jnp.float32)
    e = jnp.sum(out * qb, axis=1, keepdims=True)
    emax = jnp.max(jnp.where(onehot > 0.0, e, -1e30), axis=0,
                   keepdims=True)
    emax = jnp.where(emax > -1e29, emax, 0.0)
    ee = jnp.exp(e - jnp.sum(onehot * emax, axis=1, keepdims=True))
    den = jnp.dot(ee.T, onehot, preferred_element_type=jnp.float32)
    denb = jnp.sum(onehot * den, axis=1, keepdims=True)
    a = ee / (denb + 1e-16)
    r = lax.dot_general(onehot, a * out, (((0,), (0,)), ((), ())),
                        preferred_element_type=jnp.float32)
    h_o[...] = h
    c_o[...] = c
    qs_o[:, :_H] = h
    qs_o[:, _H:] = r


def _s2s_step(onehot, out3, h, c, q_star, Wih_T, Whh_T, bih, bhh):
    return pl.pallas_call(
        _s2s_step_kernel,
        out_shape=(jax.ShapeDtypeStruct((_G, _H), jnp.float32),
                   jax.ShapeDtypeStruct((_G, _H), jnp.float32),
                   jax.ShapeDtypeStruct((_G, 2 * _H), jnp.float32)),
    )(onehot, out3, h, c, q_star, Wih_T, Whh_T, bih, bhh)


def _mlp_kernel(qs_ref, l1w_ref, l1b_ref, l2w_ref, l2b_ref, z_ref):
    z = jnp.dot(qs_ref[...], l1w_ref[...],
                preferred_element_type=jnp.float32) + l1b_ref[...]
    z = jnp.dot(z, l2w_ref[...],
                preferred_element_type=jnp.float32) + l2b_ref[...]
    z_ref[...] = jax.nn.sigmoid(z)


def _final(parts, ms_pad, dinv, b, batch, Wih, Whh, bih, bhh,
           lin1_W, lin1_b, lin2_W, lin2_b):
    out3 = _out3(parts, ms_pad, dinv, b)
    onehot = _onehot(batch)
    h = jnp.zeros((_G, _H), jnp.float32)
    c = jnp.zeros((_G, _H), jnp.float32)
    q_star = jnp.zeros((_G, 2 * _H), jnp.float32)
    for _ in range(4):
        h, c, q_star = _s2s_step(onehot, out3, h, c, q_star,
                                 Wih.T, Whh.T, bih[None, :], bhh[None, :])
    return pl.pallas_call(
        _mlp_kernel,
        out_shape=jax.ShapeDtypeStruct((_G, 1), jnp.float32),
    )(q_star, lin1_W.T, lin1_b[None, :], lin2_W.T, lin2_b[None, :])


def kernel(x, edge_index, edge_attr, batch, emb_tables, W1, b1, W2, b2, W3, b3,
           bn_gamma, bn_beta, Wih, Whh, bih, bhh, lin1_W, lin1_b, lin2_W, lin2_b):
    del edge_attr
    src = edge_index[0].astype(jnp.int32)
    dst = edge_index[1].astype(jnp.int32)
    pad = _N + (jnp.arange(_EP - _E, dtype=jnp.int32) % (_NP - _N))
    src_w = jnp.concatenate([src, pad]).reshape(_NWORK, _NWIN, _W)
    dst_w = jnp.concatenate([dst, pad]).reshape(_NWORK, _NWIN, _W)

    deg_parts = _sc_deg(dst_w)
    emb = _emb(x.astype(jnp.int32), jnp.concatenate(emb_tables, axis=0))
    ms1, dinv = _pre(emb, deg_parts, W1)

    parts1 = _sc_conv(ms1, src_w, dst_w)
    ms2 = _post(parts1, ms1, dinv, b1, bn_gamma, bn_beta, W2)
    parts2 = _sc_conv(ms2, src_w, dst_w)
    ms3 = _post(parts2, ms2, dinv, b2, bn_gamma, bn_beta, W3)
    parts3 = _sc_conv(ms3, src_w, dst_w)

    return _final(parts3, ms3, dinv, b3, batch, Wih, Whh, bih, bhh,
                  lin1_W, lin1_b, lin2_W, lin2_b)


# speedup vs baseline: 24.4550x; 1.3766x over previous
import functools

import jax
import jax.numpy as jnp
from jax import lax
from jax.experimental import pallas as pl
from jax.experimental.pallas import tpu as pltpu
from jax.experimental.pallas import tpu_sc as plsc

_N = 10000
_NP = 10240
_E = 320000
_H = 128
_G = 256
_W = 128
_NWIN = 80
_NWORK = 32
_EP = _W * _NWIN * _NWORK
_RPS = _NP // 16
_FDIMS = (119, 4, 12, 12, 10, 6, 6, 2, 2)
_FTOT = sum(_FDIMS)

@functools.cache
def _mesh():
    return plsc.VectorSubcoreMesh(core_axis_name="core",
                                  subcore_axis_name="subcore")


_NCH = _NWIN // 2


def _sc_conv_kernel(ms_hbm, src_hbm, dst_hbm, out_hbm,
                    src_v, dst_v, rows0_v, rows1_v, acc_sh, sem0, sem1):
    cid = lax.axis_index("core")
    sid = lax.axis_index("subcore")
    wid = cid * 16 + sid

    @pl.loop(0, _W)
    def _(r):
        @pl.loop(0, _H, step=16)
        def _(c):
            rows0_v.at[pl.ds(r, 1), pl.ds(c, 16)][...] = jnp.zeros(
                (1, 16), jnp.float32)

    @pl.loop(0, _RPS // _W)
    def _(i):
        pltpu.sync_copy(rows0_v, acc_sh.at[pl.ds(sid * _RPS + i * _W, _W)])

    plsc.subcore_barrier()

    for ch in range(_NWIN // _NCH):
        pltpu.sync_copy(src_hbm.at[wid, pl.ds(ch * _NCH, _NCH)], src_v)
        pltpu.sync_copy(dst_hbm.at[wid, pl.ds(ch * _NCH, _NCH)], dst_v)
        pltpu.async_copy(ms_hbm.at[src_v.at[0]], rows0_v, sem0)

        @pl.loop(0, _NCH, step=2)
        def _(w):
            pltpu.async_copy(ms_hbm.at[src_v.at[w + 1]], rows1_v, sem1)
            pltpu.make_async_copy(ms_hbm.at[src_v.at[w]], rows0_v,
                                  sem0).wait()
            pltpu.sync_copy(rows0_v, acc_sh.at[dst_v.at[w]], add=True)

            @pl.when(w + 2 < _NCH)
            def _():
                pltpu.async_copy(ms_hbm.at[src_v.at[w + 2]], rows0_v, sem0)

            pltpu.make_async_copy(ms_hbm.at[src_v.at[w + 1]], rows1_v,
                                  sem1).wait()
            pltpu.sync_copy(rows1_v, acc_sh.at[dst_v.at[w + 1]], add=True)

    plsc.subcore_barrier()
    pltpu.sync_copy(acc_sh.at[pl.ds(sid * _RPS, _RPS)],
                    out_hbm.at[cid, pl.ds(sid * _RPS, _RPS)])


def _sc_conv(ms_pad, src_w, dst_w):
    return pl.kernel(
        _sc_conv_kernel,
        out_type=jax.ShapeDtypeStruct((2, _NP, _H), jnp.float32),
        mesh=_mesh(),
        scratch_types=[
            pltpu.VMEM((_NCH, _W), jnp.int32),
            pltpu.VMEM((_NCH, _W), jnp.int32),
            pltpu.VMEM((_W, _H), jnp.float32),
            pltpu.VMEM((_W, _H), jnp.float32),
            pltpu.VMEM_SHARED((_NP, _H), jnp.float32),
            pltpu.SemaphoreType.DMA,
            pltpu.SemaphoreType.DMA,
        ],
    )(ms_pad, src_w, dst_w)


def _sc_deg_kernel(dst_hbm, out_hbm, dst_v, ones_v, acc_sh):
    cid = lax.axis_index("core")
    sid = lax.axis_index("subcore")
    wid = cid * 16 + sid

    @pl.loop(0, _W)
    def _(r):
        ones_v.at[pl.ds(r, 1), pl.ds(0, 16)][...] = jnp.ones((1, 16),
                                                             jnp.float32)

    @pl.loop(0, _RPS // _W)
    def _(i):
        pltpu.sync_copy(ones_v, acc_sh.at[pl.ds(sid * _RPS + i * _W, _W)])

    plsc.subcore_barrier()

    pltpu.sync_copy(dst_hbm.at[wid], dst_v)

    @pl.loop(0, _NWIN)
    def _(w):
        pltpu.sync_copy(ones_v, acc_sh.at[dst_v.at[w]], add=True)

    plsc.subcore_barrier()
    pltpu.sync_copy(acc_sh.at[pl.ds(sid * _RPS, _RPS)],
                    out_hbm.at[cid, pl.ds(sid * _RPS, _RPS)])


def _sc_deg(dst_w):
    return pl.kernel(
        _sc_deg_kernel,
        out_type=jax.ShapeDtypeStruct((2, _NP, 16), jnp.float32),
        mesh=_mesh(),
        scratch_types=[
            pltpu.VMEM((_NWIN, _W), jnp.int32),
            pltpu.VMEM((_W, 16), jnp.float32),
            pltpu.VMEM_SHARED((_NP, 16), jnp.float32),
        ],
    )(dst_w)


def _emb_kernel(x_ref, t_ref, o_ref):
    cols = lax.broadcasted_iota(jnp.int32, (_N, _FTOT), 1)
    mh = jnp.zeros((_N, _FTOT), jnp.float32)
    off = 0
    for f, d in enumerate(_FDIMS):
        mh = mh + (cols == x_ref[:, f:f + 1] + off).astype(jnp.float32)
        off += d
    o_ref[...] = jnp.dot(mh, t_ref[...], preferred_element_type=jnp.float32)


def _emb(x, tables_cat):
    return pl.pallas_call(
        _emb_kernel,
        out_shape=jax.ShapeDtypeStruct((_N, _H), jnp.float32),
    )(x, tables_cat)


def _pre_kernel(emb_ref, degp_ref, w_ref, o_ref, dinv_ref):
    deg = degp_ref[0, :_N, 0:1] + degp_ref[1, :_N, 0:1] - 1.0
    dinv = lax.rsqrt(deg)
    dinv_ref[...] = dinv
    ms = jnp.dot(emb_ref[...], w_ref[...],
                 preferred_element_type=jnp.float32) * dinv
    o_ref[:_N, :] = ms
    o_ref[_N:, :] = jnp.zeros((_NP - _N, _H), jnp.float32)


def _pre(emb, deg_parts, W1):
    return pl.pallas_call(
        _pre_kernel,
        out_shape=(jax.ShapeDtypeStruct((_NP, _H), jnp.float32),
                   jax.ShapeDtypeStruct((_N, 1), jnp.float32)),
    )(emb, deg_parts, W1.T)


def _post_kernel(p_ref, ms_ref, dinv_ref, b_ref, g_ref, bt_ref, w_ref, o_ref):
    dinv = dinv_ref[...]
    s = p_ref[0, :_N, :] + p_ref[1, :_N, :] + ms_ref[:_N, :]
    h = dinv * s + b_ref[...]
    h = jnp.maximum(h, 0.0)
    mu = jnp.mean(h, axis=0, keepdims=True)
    var = jnp.mean((h - mu) ** 2, axis=0, keepdims=True)
    h = (h - mu) / jnp.sqrt(var + 1e-5) * g_ref[...] + bt_ref[...]
    ms = jnp.dot(h, w_ref[...], preferred_element_type=jnp.float32) * dinv
    o_ref[:_N, :] = ms
    o_ref[_N:, :] = jnp.zeros((_NP - _N, _H), jnp.float32)


def _post(parts, ms_pad, dinv, b, gamma, beta, W_next):
    return pl.pallas_call(
        _post_kernel,
        out_shape=jax.ShapeDtypeStruct((_NP, _H), jnp.float32),
    )(parts, ms_pad, dinv, b[None, :], gamma[None, :], beta[None, :], W_next.T)


def _out3_kernel(p_ref, ms_ref, dinv_ref, b_ref, o_ref):
    s = p_ref[0, :_N, :] + p_ref[1, :_N, :] + ms_ref[:_N, :]
    o_ref[...] = dinv_ref[...] * s + b_ref[...]


def _out3(parts, ms_pad, dinv, b):
    return pl.pallas_call(
        _out3_kernel,
        out_shape=jax.ShapeDtypeStruct((_N, _H), jnp.float32),
    )(parts, ms_pad, dinv, b[None, :])


def _onehot_kernel(batch_ref, o_ref):
    o_ref[...] = (batch_ref[...] == lax.broadcasted_iota(
        jnp.int32, (_N, _G), 1)).astype(jnp.float32)


def _onehot(batch):
    return pl.pallas_call(
        _onehot_kernel,
        out_shape=jax.ShapeDtypeStruct((_N, _G), jnp.float32),
    )(batch[:, None].astype(jnp.int32))


def _s2s_step_kernel(oh_ref, out_ref, h_ref, c_ref, qs_ref,
                     wih_ref, whh_ref, bih_ref, bhh_ref,
                     h_o, c_o, qs_o):
    onehot = oh_ref[...]
    out = out_ref[...]
    g = (jnp.dot(qs_ref[...], wih_ref[...], preferred_element_type=jnp.float32)
         + bih_ref[...]
         + jnp.dot(h_ref[...], whh_ref[...], preferred_element_type=jnp.float32)
         + bhh_ref[...])
    i_g, f_g, g_g, o_g = jnp.split(g, 4, axis=-1)
    c = jax.nn.sigmoid(f_g) * c_ref[...] + jax.nn.sigmoid(i_g) * jnp.tanh(g_g)
    h = jax.nn.sigmoid(o_g) * jnp.tanh(c)
    qb = jnp.dot(onehot, h, preferred_element_type=jnp.float32)
    e = jnp.sum(out * qb, axis=1, keepdims=True)
    emax = jnp.max(jnp.where(onehot > 0.0, e, -1e30), axis=0,
                   keepdims=True)
    emax = jnp.where(emax > -1e29, emax, 0.0)
    ee = jnp.exp(e - jnp.sum(onehot * emax, axis=1, keepdims=True))
    den = jnp.dot(ee.T, onehot, preferred_element_type=jnp.float32)
    denb = jnp.sum(onehot * den, axis=1, keepdims=True)
    a = ee / (denb + 1e-16)
    r = lax.dot_general(onehot, a * out, (((0,), (0,)), ((), ())),
                        preferred_element_type=jnp.float32)
    h_o[...] = h
    c_o[...] = c
    qs_o[:, :_H] = h
    qs_o[:, _H:] = r


def _s2s_step(onehot, out3, h, c, q_star, Wih_T, Whh_T, bih, bhh):
    return pl.pallas_call(
        _s2s_step_kernel,
        out_shape=(jax.ShapeDtypeStruct((_G, _H), jnp.float32),
                   jax.ShapeDtypeStruct((_G, _H), jnp.float32),
                   jax.ShapeDtypeStruct((_G, 2 * _H), jnp.float32)),
    )(onehot, out3, h, c, q_star, Wih_T, Whh_T, bih, bhh)


def _mlp_kernel(qs_ref, l1w_ref, l1b_ref, l2w_ref, l2b_ref, z_ref):
    z = jnp.dot(qs_ref[...], l1w_ref[...],
                preferred_element_type=jnp.float32) + l1b_ref[...]
    z = jnp.dot(z, l2w_ref[...],
                preferred_element_type=jnp.float32) + l2b_ref[...]
    z_ref[...] = jax.nn.sigmoid(z)


def _final(parts, ms_pad, dinv, b, batch, Wih, Whh, bih, bhh,
           lin1_W, lin1_b, lin2_W, lin2_b):
    out3 = _out3(parts, ms_pad, dinv, b)
    onehot = _onehot(batch)
    h = jnp.zeros((_G, _H), jnp.float32)
    c = jnp.zeros((_G, _H), jnp.float32)
    q_star = jnp.zeros((_G, 2 * _H), jnp.float32)
    for _ in range(4):
        h, c, q_star = _s2s_step(onehot, out3, h, c, q_star,
                                 Wih.T, Whh.T, bih[None, :], bhh[None, :])
    return pl.pallas_call(
        _mlp_kernel,
        out_shape=jax.ShapeDtypeStruct((_G, 1), jnp.float32),
    )(q_star, lin1_W.T, lin1_b[None, :], lin2_W.T, lin2_b[None, :])


def kernel(x, edge_index, edge_attr, batch, emb_tables, W1, b1, W2, b2, W3, b3,
           bn_gamma, bn_beta, Wih, Whh, bih, bhh, lin1_W, lin1_b, lin2_W, lin2_b):
    del edge_attr
    src = edge_index[0].astype(jnp.int32)
    dst = edge_index[1].astype(jnp.int32)
    pad = _N + (jnp.arange(_EP - _E, dtype=jnp.int32) % (_NP - _N))
    src_w = jnp.concatenate([src, pad]).reshape(_NWORK, _NWIN, _W)
    dst_w = jnp.concatenate([dst, pad]).reshape(_NWORK, _NWIN, _W)

    deg_parts = _sc_deg(dst_w)
    emb = _emb(x.astype(jnp.int32), jnp.concatenate(emb_tables, axis=0))
    ms1, dinv = _pre(emb, deg_parts, W1)

    parts1 = _sc_conv(ms1, src_w, dst_w)
    ms2 = _post(parts1, ms1, dinv, b1, bn_gamma, bn_beta, W2)
    parts2 = _sc_conv(ms2, src_w, dst_w)
    ms3 = _post(parts2, ms2, dinv, b2, bn_gamma, bn_beta, W3)
    parts3 = _sc_conv(ms3, src_w, dst_w)

    return _final(parts3, ms3, dinv, b3, batch, Wih, Whh, bih, bhh,
                  lin1_W, lin1_b, lin2_W, lin2_b)


# speedup vs baseline: 25.6274x; 1.0479x over previous
import functools

import jax
import jax.numpy as jnp
from jax import lax
from jax.experimental import pallas as pl
from jax.experimental.pallas import tpu as pltpu
from jax.experimental.pallas import tpu_sc as plsc

_N = 10000
_NP = 10240
_E = 320000
_H = 128
_G = 256
_W = 128
_NWIN = 80
_NWORK = 32
_EP = _W * _NWIN * _NWORK
_RPS = _NP // 16
_FDIMS = (119, 4, 12, 12, 10, 6, 6, 2, 2)
_FTOT = sum(_FDIMS)

@functools.cache
def _mesh():
    return plsc.VectorSubcoreMesh(core_axis_name="core",
                                  subcore_axis_name="subcore")


_NCH = _NWIN // 2


def _sc_conv_kernel(ms_hbm, src_hbm, dst_hbm, out_hbm,
                    src_v, dst_v, rows0_v, rows1_v, acc_sh, sem0, sem1):
    cid = lax.axis_index("core")
    sid = lax.axis_index("subcore")
    wid = cid * 16 + sid

    @pl.loop(0, _W)
    def _(r):
        @pl.loop(0, _H, step=16)
        def _(c):
            rows0_v.at[pl.ds(r, 1), pl.ds(c, 16)][...] = jnp.zeros(
                (1, 16), jnp.float32)

    @pl.loop(0, _RPS // _W)
    def _(i):
        pltpu.sync_copy(rows0_v, acc_sh.at[pl.ds(sid * _RPS + i * _W, _W)])

    plsc.subcore_barrier()

    for ch in range(_NWIN // _NCH):
        pltpu.sync_copy(src_hbm.at[wid, pl.ds(ch * _NCH, _NCH)], src_v)
        pltpu.sync_copy(dst_hbm.at[wid, pl.ds(ch * _NCH, _NCH)], dst_v)
        pltpu.async_copy(ms_hbm.at[src_v.at[0]], rows0_v, sem0)

        @pl.loop(0, _NCH, step=2)
        def _(w):
            pltpu.async_copy(ms_hbm.at[src_v.at[w + 1]], rows1_v, sem1)
            pltpu.make_async_copy(ms_hbm.at[src_v.at[w]], rows0_v,
                                  sem0).wait()
            pltpu.sync_copy(rows0_v, acc_sh.at[dst_v.at[w]], add=True)

            @pl.when(w + 2 < _NCH)
            def _():
                pltpu.async_copy(ms_hbm.at[src_v.at[w + 2]], rows0_v, sem0)

            pltpu.make_async_copy(ms_hbm.at[src_v.at[w + 1]], rows1_v,
                                  sem1).wait()
            pltpu.sync_copy(rows1_v, acc_sh.at[dst_v.at[w + 1]], add=True)

    plsc.subcore_barrier()
    pltpu.sync_copy(acc_sh.at[pl.ds(sid * _RPS, _RPS)],
                    out_hbm.at[cid, pl.ds(sid * _RPS, _RPS)])


def _sc_conv(ms_pad, src_w, dst_w):
    return pl.kernel(
        _sc_conv_kernel,
        out_type=jax.ShapeDtypeStruct((2, _NP, _H), jnp.float32),
        mesh=_mesh(),
        scratch_types=[
            pltpu.VMEM((_NCH, _W), jnp.int32),
            pltpu.VMEM((_NCH, _W), jnp.int32),
            pltpu.VMEM((_W, _H), jnp.float32),
            pltpu.VMEM((_W, _H), jnp.float32),
            pltpu.VMEM_SHARED((_NP, _H), jnp.float32),
            pltpu.SemaphoreType.DMA,
            pltpu.SemaphoreType.DMA,
        ],
    )(ms_pad, src_w, dst_w)


def _sc_deg_kernel(dst_hbm, out_hbm, dst_v, ones_v, acc_sh):
    cid = lax.axis_index("core")
    sid = lax.axis_index("subcore")
    wid = cid * 16 + sid

    @pl.loop(0, _W)
    def _(r):
        ones_v.at[pl.ds(r, 1), pl.ds(0, 16)][...] = jnp.ones((1, 16),
                                                             jnp.float32)

    @pl.loop(0, _RPS // _W)
    def _(i):
        pltpu.sync_copy(ones_v, acc_sh.at[pl.ds(sid * _RPS + i * _W, _W)])

    plsc.subcore_barrier()

    pltpu.sync_copy(dst_hbm.at[wid], dst_v)

    @pl.loop(0, _NWIN)
    def _(w):
        pltpu.sync_copy(ones_v, acc_sh.at[dst_v.at[w]], add=True)

    plsc.subcore_barrier()
    pltpu.sync_copy(acc_sh.at[pl.ds(sid * _RPS, _RPS)],
                    out_hbm.at[cid, pl.ds(sid * _RPS, _RPS)])


def _sc_deg(dst_w):
    return pl.kernel(
        _sc_deg_kernel,
        out_type=jax.ShapeDtypeStruct((2, _NP, 16), jnp.float32),
        mesh=_mesh(),
        scratch_types=[
            pltpu.VMEM((_NWIN, _W), jnp.int32),
            pltpu.VMEM((_W, 16), jnp.float32),
            pltpu.VMEM_SHARED((_NP, 16), jnp.float32),
        ],
    )(dst_w)


def _emb_pre_kernel(x_ref, t_ref, degp_ref, w_ref, o_ref, dinv_ref):
    cols = lax.broadcasted_iota(jnp.int32, (_N, _FTOT), 1)
    mh = jnp.zeros((_N, _FTOT), jnp.float32)
    off = 0
    for f, d in enumerate(_FDIMS):
        mh = mh + (cols == x_ref[:, f:f + 1] + off).astype(jnp.float32)
        off += d
    emb = jnp.dot(mh, t_ref[...], preferred_element_type=jnp.float32)
    deg = degp_ref[0, :_N, 0:1] + degp_ref[1, :_N, 0:1] - 1.0
    dinv = lax.rsqrt(deg)
    dinv_ref[...] = dinv
    ms = jnp.dot(emb, w_ref[...], preferred_element_type=jnp.float32) * dinv
    o_ref[:_N, :] = ms
    o_ref[_N:, :] = jnp.zeros((_NP - _N, _H), jnp.float32)


def _emb_pre(x, tables_cat, deg_parts, W1):
    return pl.pallas_call(
        _emb_pre_kernel,
        out_shape=(jax.ShapeDtypeStruct((_NP, _H), jnp.float32),
                   jax.ShapeDtypeStruct((_N, 1), jnp.float32)),
    )(x, tables_cat, deg_parts, W1.T)


def _post_kernel(p_ref, ms_ref, dinv_ref, b_ref, g_ref, bt_ref, w_ref, o_ref):
    dinv = dinv_ref[...]
    s = p_ref[0, :_N, :] + p_ref[1, :_N, :] + ms_ref[:_N, :]
    h = dinv * s + b_ref[...]
    h = jnp.maximum(h, 0.0)
    mu = jnp.mean(h, axis=0, keepdims=True)
    var = jnp.mean((h - mu) ** 2, axis=0, keepdims=True)
    h = (h - mu) / jnp.sqrt(var + 1e-5) * g_ref[...] + bt_ref[...]
    ms = jnp.dot(h, w_ref[...], preferred_element_type=jnp.float32) * dinv
    o_ref[:_N, :] = ms
    o_ref[_N:, :] = jnp.zeros((_NP - _N, _H), jnp.float32)


def _post(parts, ms_pad, dinv, b, gamma, beta, W_next):
    return pl.pallas_call(
        _post_kernel,
        out_shape=jax.ShapeDtypeStruct((_NP, _H), jnp.float32),
    )(parts, ms_pad, dinv, b[None, :], gamma[None, :], beta[None, :], W_next.T)


def _out3_kernel(p_ref, ms_ref, dinv_ref, b_ref, batch_ref, o_ref, oh_ref):
    s = p_ref[0, :_N, :] + p_ref[1, :_N, :] + ms_ref[:_N, :]
    o_ref[...] = dinv_ref[...] * s + b_ref[...]
    oh_ref[...] = (batch_ref[...] == lax.broadcasted_iota(
        jnp.int32, (_N, _G), 1)).astype(jnp.float32)


def _s2s_kernel(oh_ref, out_ref,
                wih_ref, whh_ref, bih_ref, bhh_ref,
                l1w_ref, l1b_ref, l2w_ref, l2b_ref, z_ref):
    out = out_ref[...]
    onehot = oh_ref[...]

    h = jnp.zeros((_G, _H), jnp.float32)
    c = jnp.zeros((_G, _H), jnp.float32)
    q_star = jnp.zeros((_G, 2 * _H), jnp.float32)
    for _ in range(4):
        g = (jnp.dot(q_star, wih_ref[...],
                     preferred_element_type=jnp.float32) + bih_ref[...]
             + jnp.dot(h, whh_ref[...],
                       preferred_element_type=jnp.float32) + bhh_ref[...])
        i_g, f_g, g_g, o_g = jnp.split(g, 4, axis=-1)
        c = jax.nn.sigmoid(f_g) * c + jax.nn.sigmoid(i_g) * jnp.tanh(g_g)
        h = jax.nn.sigmoid(o_g) * jnp.tanh(c)
        qb = jnp.dot(onehot, h, preferred_element_type=jnp.float32)
        e = jnp.sum(out * qb, axis=1, keepdims=True)
        emax = jnp.max(jnp.where(onehot > 0.0, e, -1e30), axis=0,
                       keepdims=True)
        emax = jnp.where(emax > -1e29, emax, 0.0)
        ee = jnp.exp(e - jnp.sum(onehot * emax, axis=1, keepdims=True))
        den = jnp.dot(ee.T, onehot, preferred_element_type=jnp.float32)
        denb = jnp.sum(onehot * den, axis=1, keepdims=True)
        a = ee / (denb + 1e-16)
        r = lax.dot_general(onehot, a * out, (((0,), (0,)), ((), ())),
                            preferred_element_type=jnp.float32)
        q_star = jnp.concatenate([h, r], axis=1)
    z = jnp.dot(q_star, l1w_ref[...],
                preferred_element_type=jnp.float32) + l1b_ref[...]
    z = jnp.dot(z, l2w_ref[...],
                preferred_element_type=jnp.float32) + l2b_ref[...]
    z_ref[...] = jax.nn.sigmoid(z)


def _final(parts, ms_pad, dinv, b, batch, Wih, Whh, bih, bhh,
           lin1_W, lin1_b, lin2_W, lin2_b):
    out3, onehot = pl.pallas_call(
        _out3_kernel,
        out_shape=(jax.ShapeDtypeStruct((_N, _H), jnp.float32),
                   jax.ShapeDtypeStruct((_N, _G), jnp.float32)),
    )(parts, ms_pad, dinv, b[None, :], batch[:, None].astype(jnp.int32))
    return pl.pallas_call(
        _s2s_kernel,
        out_shape=jax.ShapeDtypeStruct((_G, 1), jnp.float32),
    )(onehot, out3,
      Wih.T, Whh.T, bih[None, :], bhh[None, :],
      lin1_W.T, lin1_b[None, :], lin2_W.T, lin2_b[None, :])


def kernel(x, edge_index, edge_attr, batch, emb_tables, W1, b1, W2, b2, W3, b3,
           bn_gamma, bn_beta, Wih, Whh, bih, bhh, lin1_W, lin1_b, lin2_W, lin2_b):
    del edge_attr
    src = edge_index[0].astype(jnp.int32)
    dst = edge_index[1].astype(jnp.int32)
    pad = _N + (jnp.arange(_EP - _E, dtype=jnp.int32) % (_NP - _N))
    src_w = jnp.concatenate([src, pad]).reshape(_NWORK, _NWIN, _W)
    dst_w = jnp.concatenate([dst, pad]).reshape(_NWORK, _NWIN, _W)

    deg_parts = _sc_deg(dst_w)
    ms1, dinv = _emb_pre(x.astype(jnp.int32),
                         jnp.concatenate(emb_tables, axis=0), deg_parts, W1)

    parts1 = _sc_conv(ms1, src_w, dst_w)
    ms2 = _post(parts1, ms1, dinv, b1, bn_gamma, bn_beta, W2)
    parts2 = _sc_conv(ms2, src_w, dst_w)
    ms3 = _post(parts2, ms2, dinv, b2, bn_gamma, bn_beta, W3)
    parts3 = _sc_conv(ms3, src_w, dst_w)

    return _final(parts3, ms3, dinv, b3, batch, Wih, Whh, bih, bhh,
                  lin1_W, lin1_b, lin2_W, lin2_b)
